# baseline (speedup 1.0000x reference)
import functools

import jax
import jax.numpy as jnp
from jax import lax
from jax.experimental import pallas as pl
from jax.experimental.pallas import tpu as pltpu
from jax.experimental.pallas import tpu_sc as plsc

_C0 = 0.28209479177387814

_NC = 2
_NS = 16
_NW = _NC * _NS

_GB = 128

_NPAD = 50176


_LOG1P_C = (3.910905554960209e-08, 0.9999936302585147, -0.49982549864347925,
            0.33144665224343317, -0.23943337074600235, 0.16499812983410006,
            -0.09229041738055285, 0.03426459995555095, -0.006006605050865348)


def _softplus(x):
    u = jnp.exp(-jnp.abs(x))
    p = jnp.float32(_LOG1P_C[-1])
    for coef in _LOG1P_C[-2::-1]:
        p = p * u + jnp.float32(coef)
    return jnp.maximum(x, 0.0) + p


def _edge_body(rlo_ref, rhi_ref, wcat_ref, bcat_ref, v_ref, bv_ref, out_ref):
    nbr = wcat_ref.shape[0]
    m_slots = rlo_ref.shape[1] // nbr
    for q, ref in ((0, rlo_ref), (1, rhi_ref)):
        for m in range(m_slots):
            a = jnp.dot(ref[:, nbr * m:nbr * (m + 1)], wcat_ref[...],
                        preferred_element_type=jnp.float32) + bcat_ref[...]
            a = _softplus(a)
            out_ref[:, 64 * q + 4 * m:64 * q + 4 * m + 4] = jnp.dot(
                a, v_ref[...], preferred_element_type=jnp.float32) + bv_ref[...]


def _edge_scalars(radial256, wcat, bcat, v, bv, block):
    n, w = radial256.shape
    half_blocks = (n // 2) // block
    return pl.pallas_call(
        _edge_body,
        grid=(half_blocks,),
        in_specs=[
            pl.BlockSpec((block, w), lambda i: (i, 0)),
            pl.BlockSpec((block, w), lambda i: (i + half_blocks, 0)),
            pl.BlockSpec(wcat.shape, lambda i: (0, 0)),
            pl.BlockSpec(bcat.shape, lambda i: (0, 0)),
            pl.BlockSpec(v.shape, lambda i: (0, 0)),
            pl.BlockSpec(bv.shape, lambda i: (0, 0)),
        ],
        out_specs=pl.BlockSpec((block, 128), lambda i: (i, 0)),
        out_shape=jax.ShapeDtypeStruct((n // 2, 128), jnp.float32),
    )(radial256, radial256, wcat, bcat, v, bv)


_CHUNK_R = 100
_BIG_W = 30
_BIG_R = 800
_SMALL_R = 500
_SUB = 25


def _sc_scatter(nbr_idx, vals, zeros):
    m = nbr_idx.shape[1]
    half = vals.shape[0]
    acc_w = _NPAD * 4
    words_t = acc_w // _NS

    mesh = plsc.VectorSubcoreMesh(core_axis_name="c", subcore_axis_name="s")

    @functools.partial(
        pl.kernel,
        out_type=jax.ShapeDtypeStruct((_NC, acc_w), jnp.float32),
        mesh=mesh,
        scratch_types=[
            pltpu.VMEM((_CHUNK_R, m), jnp.int32),
            pltpu.VMEM((_CHUNK_R, m), jnp.int32),
            pltpu.VMEM((_CHUNK_R, 128), jnp.int32),
            pltpu.VMEM((_CHUNK_R, 128), jnp.float32),
            pltpu.VMEM_SHARED((acc_w,), jnp.float32),
            pltpu.SemaphoreType.DMA,
        ],
        compiler_params=pltpu.CompilerParams(use_tc_tiling_on_sc=False),
    )
    def scatter(idx_hbm, vals_hbm, zeros_hbm, out_hbm,
                idx_lo, idx_hi, idxw, vals_v, acc, sem):
        c = lax.axis_index("c")
        s = lax.axis_index("s")
        wid = s * _NC + c

        pltpu.sync_copy(zeros_hbm.at[pl.ds(s * words_t, words_t)],
                        acc.at[pl.ds(s * words_t, words_t)])
        plsc.subcore_barrier()

        base = jnp.where(wid < _BIG_W, wid * _BIG_R,
                         _BIG_W * _BIG_R + (wid - _BIG_W) * _SMALL_R)
        n_chunks = jnp.where(wid < _BIG_W, _BIG_R // _CHUNK_R,
                             _SMALL_R // _CHUNK_R)

        lanes = lax.iota(jnp.int32, 16)
        mod4 = lax.rem(lanes, jnp.int32(4))
        div4 = lax.div(lanes, jnp.int32(4))
        dn = lax.GatherDimensionNumbers(
            offset_dims=(), collapsed_slice_dims=(0,), start_index_map=(0,))
        perms = [(div4 + 4 * j)[:, None] for j in range(4)]

        @pl.loop(0, n_chunks)
        def _(k):
            r0 = base + k * _CHUNK_R
            pltpu.sync_copy(idx_hbm.at[pl.ds(r0, _CHUNK_R)], idx_lo)
            pltpu.sync_copy(idx_hbm.at[pl.ds(half + r0, _CHUNK_R)], idx_hi)
            pltpu.sync_copy(vals_hbm.at[pl.ds(r0, _CHUNK_R)], vals_v)

            @pl.loop(0, _CHUNK_R)
            def _(r):
                row_lo = idx_lo[r, :]
                row_hi = idx_hi[r, :]
                for j in range(4):
                    seg = lax.gather(row_lo, perms[j], dn, (1,),
                                     mode=lax.GatherScatterMode.PROMISE_IN_BOUNDS)
                    idxw[r, 16 * j:16 * j + 16] = seg * 4 + mod4
                for j in range(4):
                    seg = lax.gather(row_hi, perms[j], dn, (1,),
                                     mode=lax.GatherScatterMode.PROMISE_IN_BOUNDS)
                    idxw[r, 64 + 16 * j:64 + 16 * j + 16] = seg * 4 + mod4

            for sub in range(_CHUNK_R // _SUB):
                @pl.loop(sub * _SUB, (sub + 1) * _SUB)
                def _(r):
                    pltpu.async_copy(vals_v.at[r], acc.at[idxw.at[r]], sem,
                                     add=True)

                @pl.loop(0, _SUB)
                def _(r):
                    pltpu.make_async_copy(vals_v.at[0], acc.at[idxw.at[0]],
                                          sem).wait()

        plsc.subcore_barrier()
        pltpu.sync_copy(acc.at[pl.ds(s * words_t, words_t)],
                        out_hbm.at[c, pl.ds(s * words_t, words_t)])

    return scatter(nbr_idx, vals, zeros)


def _dense_body(atom_ref, s0_ref, s1_ref, wemb_ref, bemb_ref,
                tpw_ref, pool_ref, wfc_ref, bfc_ref, wout_ref, bout_ref,
                h_ref, o_ref):
    x = jnp.dot(atom_ref[...], wemb_ref[...],
                preferred_element_type=jnp.float32) + bemb_ref[...]
    s = s0_ref[...] + s1_ref[...]
    inv = 1.0 / jnp.maximum(s[:, 3:4], 1.0)
    for j in range(3):
        x = jnp.dot(x * (s[:, j:j + 1] * inv), tpw_ref[j],
                    preferred_element_type=jnp.float32) * 0.125
    crys = jnp.dot(pool_ref[...], x, preferred_element_type=jnp.float32)
    h = _softplus(jnp.dot(crys, wfc_ref[...],
                          preferred_element_type=jnp.float32) + bfc_ref[...])
    h_ref[0] = h
    o_ref[0] = jnp.dot(h, wout_ref[...],
                       preferred_element_type=jnp.float32) + bout_ref[...]


def _dense(atom_fea, s0, s1, wemb, bemb, tpw, pool, wfc, bfc, wout, bout,
           rows, n_cry, n_blocks):
    cpb = n_cry // n_blocks
    h3, o3 = pl.pallas_call(
        _dense_body,
        grid=(n_blocks,),
        in_specs=[
            pl.BlockSpec((rows, atom_fea.shape[1]), lambda i: (i, 0)),
            pl.BlockSpec((rows, 4), lambda i: (i, 0)),
            pl.BlockSpec((rows, 4), lambda i: (i, 0)),
            pl.BlockSpec(wemb.shape, lambda i: (0, 0)),
            pl.BlockSpec(bemb.shape, lambda i: (0, 0)),
            pl.BlockSpec(tpw.shape, lambda i: (0, 0, 0)),
            pl.BlockSpec(pool.shape, lambda i: (0, 0)),
            pl.BlockSpec(wfc.shape, lambda i: (0, 0)),
            pl.BlockSpec(bfc.shape, lambda i: (0, 0)),
            pl.BlockSpec(wout.shape, lambda i: (0, 0)),
            pl.BlockSpec(bout.shape, lambda i: (0, 0)),
        ],
        out_specs=[
            pl.BlockSpec((1, cpb, wfc.shape[1]), lambda i: (i, 0, 0)),
            pl.BlockSpec((1, cpb, 1), lambda i: (i, 0, 0)),
        ],
        out_shape=[
            jax.ShapeDtypeStruct((n_blocks, cpb, wfc.shape[1]), jnp.float32),
            jax.ShapeDtypeStruct((n_blocks, cpb, 1), jnp.float32),
        ],
    )(atom_fea, s0, s1, wemb, bemb, tpw, pool, wfc, bfc, wout, bout)
    return h3.reshape(n_cry, wfc.shape[1]), o3.reshape(n_cry, 1)


def kernel(atom_fea, nbr_fea, nbr_idx, crystal_atom_idx, pos, W_emb, b_emb,
           Wr1_0, br1_0, Wr2_0, br2_0, tpw_0,
           Wr1_1, br1_1, Wr2_1, br2_1, tpw_1,
           Wr1_2, br1_2, Wr2_2, br2_2, tpw_2,
           W_fc, b_fc, W_out, b_out):
    n, m, nbr = nbr_fea.shape
    e = n * m
    n_cry, apc = crystal_atom_idx.shape

    wcat = jnp.concatenate([Wr1_0, Wr1_1, Wr1_2], axis=1)
    bcat = jnp.concatenate([br1_0, br1_1, br1_2])[None, :]
    eye = jnp.eye(3, dtype=jnp.float32)
    v = jnp.concatenate([
        _C0 * Wr2_0[:, :1] * eye[0], _C0 * Wr2_1[:, :1] * eye[1],
        _C0 * Wr2_2[:, :1] * eye[2]], axis=0)
    v = jnp.concatenate([v, jnp.zeros((3 * nbr, 1), jnp.float32)], axis=1)
    bv = jnp.stack([_C0 * br2_0[0], _C0 * br2_1[0], _C0 * br2_2[0],
                    jnp.float32(1.0)])[None, :]

    radial256 = nbr_fea.reshape(n, m * nbr)
    vals = _edge_scalars(radial256, wcat, bcat, v, bv, block=1000)

    acc = _sc_scatter(nbr_idx, vals, jnp.zeros((_NPAD * 4,), jnp.float32))
    s_pair = acc[:, :n * 4].reshape(2, n, 4)
    s0 = s_pair[0]
    s1 = s_pair[1]

    rows = 2000
    n_blocks = n // rows
    cpb = rows // apc
    pool = (jnp.repeat(jnp.eye(cpb, dtype=jnp.float32), apc, axis=1)
            * (1.0 / apc))
    tpw = jnp.stack([tpw_0, tpw_1, tpw_2])
    h, out = _dense(atom_fea, s0, s1, W_emb, b_emb[None, :], tpw, pool,
                    W_fc, b_fc[None, :], W_out, b_out[None, :],
                    rows, n_cry, n_blocks)
    return (out, h)


# speedup vs baseline: 1.1068x; 1.1068x over previous
import functools

import jax
import jax.numpy as jnp
from jax import lax
from jax.experimental import pallas as pl
from jax.experimental.pallas import tpu as pltpu
from jax.experimental.pallas import tpu_sc as plsc

_C0 = 0.28209479177387814

_NC = 2
_NS = 16
_NW = _NC * _NS

_GB = 128

_NPAD = 50176


def _softplus(x):
    return jnp.maximum(x, 0.0) + jnp.log1p(jnp.exp(-jnp.abs(x)))


def _edge_body(rlo_ref, rhi_ref, wcat_ref, bcat_ref, v_ref, bv_ref, out_ref):
    nbr = wcat_ref.shape[0] // 2
    m_slots = rlo_ref.shape[1] // nbr
    for m in range(m_slots):
        x2 = jnp.concatenate(
            [rlo_ref[:, nbr * m:nbr * (m + 1)],
             rhi_ref[:, nbr * m:nbr * (m + 1)]], axis=1)
        a = jnp.dot(x2, wcat_ref[...],
                    preferred_element_type=jnp.float32) + bcat_ref[...]
        a = _softplus(a)
        p = jnp.dot(a, v_ref[...],
                    preferred_element_type=jnp.float32) + bv_ref[...]
        out_ref[:, 4 * m:4 * m + 4] = p[:, 0:4]
        out_ref[:, 64 + 4 * m:64 + 4 * m + 4] = p[:, 4:8]


def _edge_scalars(radial256, wcat, bcat, v, bv, block):
    n, w = radial256.shape
    half_blocks = (n // 2) // block
    return pl.pallas_call(
        _edge_body,
        grid=(half_blocks,),
        in_specs=[
            pl.BlockSpec((block, w), lambda i: (i, 0)),
            pl.BlockSpec((block, w), lambda i: (i + half_blocks, 0)),
            pl.BlockSpec(wcat.shape, lambda i: (0, 0)),
            pl.BlockSpec(bcat.shape, lambda i: (0, 0)),
            pl.BlockSpec(v.shape, lambda i: (0, 0)),
            pl.BlockSpec(bv.shape, lambda i: (0, 0)),
        ],
        out_specs=pl.BlockSpec((block, 128), lambda i: (i, 0)),
        out_shape=jax.ShapeDtypeStruct((n // 2, 128), jnp.float32),
    )(radial256, radial256, wcat, bcat, v, bv)


_CHUNK_R = 100
_BIG_W = 30
_BIG_R = 800
_SMALL_R = 500
_SUB = 25


def _sc_scatter(nbr_idx, vals, zeros):
    m = nbr_idx.shape[1]
    half = vals.shape[0]
    acc_w = _NPAD * 4
    words_t = acc_w // _NS

    mesh = plsc.VectorSubcoreMesh(core_axis_name="c", subcore_axis_name="s")

    @functools.partial(
        pl.kernel,
        out_type=jax.ShapeDtypeStruct((_NC, acc_w), jnp.float32),
        mesh=mesh,
        scratch_types=[
            pltpu.VMEM((_CHUNK_R, m), jnp.int32),
            pltpu.VMEM((_CHUNK_R, m), jnp.int32),
            pltpu.VMEM((_CHUNK_R, 128), jnp.int32),
            pltpu.VMEM((_CHUNK_R, 128), jnp.float32),
            pltpu.VMEM_SHARED((acc_w,), jnp.float32),
            pltpu.SemaphoreType.DMA,
        ],
        compiler_params=pltpu.CompilerParams(use_tc_tiling_on_sc=False),
    )
    def scatter(idx_hbm, vals_hbm, zeros_hbm, out_hbm,
                idx_lo, idx_hi, idxw, vals_v, acc, sem):
        c = lax.axis_index("c")
        s = lax.axis_index("s")
        wid = s * _NC + c

        pltpu.sync_copy(zeros_hbm.at[pl.ds(s * words_t, words_t)],
                        acc.at[pl.ds(s * words_t, words_t)])
        plsc.subcore_barrier()

        base = jnp.where(wid < _BIG_W, wid * _BIG_R,
                         _BIG_W * _BIG_R + (wid - _BIG_W) * _SMALL_R)
        n_chunks = jnp.where(wid < _BIG_W, _BIG_R // _CHUNK_R,
                             _SMALL_R // _CHUNK_R)

        lanes = lax.iota(jnp.int32, 16)
        mod4 = lax.rem(lanes, jnp.int32(4))
        div4 = lax.div(lanes, jnp.int32(4))
        dn = lax.GatherDimensionNumbers(
            offset_dims=(), collapsed_slice_dims=(0,), start_index_map=(0,))
        perms = [(div4 + 4 * j)[:, None] for j in range(4)]

        @pl.loop(0, n_chunks)
        def _(k):
            r0 = base + k * _CHUNK_R
            pltpu.sync_copy(idx_hbm.at[pl.ds(r0, _CHUNK_R)], idx_lo)
            pltpu.sync_copy(idx_hbm.at[pl.ds(half + r0, _CHUNK_R)], idx_hi)
            pltpu.sync_copy(vals_hbm.at[pl.ds(r0, _CHUNK_R)], vals_v)

            @pl.loop(0, _CHUNK_R)
            def _(r):
                row_lo = idx_lo[r, :]
                row_hi = idx_hi[r, :]
                for j in range(4):
                    seg = lax.gather(row_lo, perms[j], dn, (1,),
                                     mode=lax.GatherScatterMode.PROMISE_IN_BOUNDS)
                    idxw[r, 16 * j:16 * j + 16] = seg * 4 + mod4
                for j in range(4):
                    seg = lax.gather(row_hi, perms[j], dn, (1,),
                                     mode=lax.GatherScatterMode.PROMISE_IN_BOUNDS)
                    idxw[r, 64 + 16 * j:64 + 16 * j + 16] = seg * 4 + mod4

            for sub in range(_CHUNK_R // _SUB):
                @pl.loop(sub * _SUB, (sub + 1) * _SUB)
                def _(r):
                    pltpu.async_copy(vals_v.at[r], acc.at[idxw.at[r]], sem,
                                     add=True)

                @pl.loop(0, _SUB)
                def _(r):
                    pltpu.make_async_copy(vals_v.at[0], acc.at[idxw.at[0]],
                                          sem).wait()

        plsc.subcore_barrier()
        pltpu.sync_copy(acc.at[pl.ds(s * words_t, words_t)],
                        out_hbm.at[c, pl.ds(s * words_t, words_t)])

    return scatter(nbr_idx, vals, zeros)


def _dense_body(atom_ref, s0_ref, s1_ref, wemb_ref, bemb_ref,
                tpw_ref, pool_ref, wfc_ref, bfc_ref, wout_ref, bout_ref,
                h_ref, o_ref):
    x = jnp.dot(atom_ref[...], wemb_ref[...],
                preferred_element_type=jnp.float32) + bemb_ref[...]
    s = s0_ref[...] + s1_ref[...]
    inv = 1.0 / jnp.maximum(s[:, 3:4], 1.0)
    for j in range(3):
        x = jnp.dot(x * (s[:, j:j + 1] * inv), tpw_ref[j],
                    preferred_element_type=jnp.float32) * 0.125
    crys = jnp.dot(pool_ref[...], x, preferred_element_type=jnp.float32)
    h = _softplus(jnp.dot(crys, wfc_ref[...],
                          preferred_element_type=jnp.float32) + bfc_ref[...])
    h_ref[0] = h
    o_ref[0] = jnp.dot(h, wout_ref[...],
                       preferred_element_type=jnp.float32) + bout_ref[...]


def _dense(atom_fea, s0, s1, wemb, bemb, tpw, pool, wfc, bfc, wout, bout,
           rows, n_cry, n_blocks):
    cpb = n_cry // n_blocks
    h3, o3 = pl.pallas_call(
        _dense_body,
        grid=(n_blocks,),
        in_specs=[
            pl.BlockSpec((rows, atom_fea.shape[1]), lambda i: (i, 0)),
            pl.BlockSpec((rows, 4), lambda i: (i, 0)),
            pl.BlockSpec((rows, 4), lambda i: (i, 0)),
            pl.BlockSpec(wemb.shape, lambda i: (0, 0)),
            pl.BlockSpec(bemb.shape, lambda i: (0, 0)),
            pl.BlockSpec(tpw.shape, lambda i: (0, 0, 0)),
            pl.BlockSpec(pool.shape, lambda i: (0, 0)),
            pl.BlockSpec(wfc.shape, lambda i: (0, 0)),
            pl.BlockSpec(bfc.shape, lambda i: (0, 0)),
            pl.BlockSpec(wout.shape, lambda i: (0, 0)),
            pl.BlockSpec(bout.shape, lambda i: (0, 0)),
        ],
        out_specs=[
            pl.BlockSpec((1, cpb, wfc.shape[1]), lambda i: (i, 0, 0)),
            pl.BlockSpec((1, cpb, 1), lambda i: (i, 0, 0)),
        ],
        out_shape=[
            jax.ShapeDtypeStruct((n_blocks, cpb, wfc.shape[1]), jnp.float32),
            jax.ShapeDtypeStruct((n_blocks, cpb, 1), jnp.float32),
        ],
    )(atom_fea, s0, s1, wemb, bemb, tpw, pool, wfc, bfc, wout, bout)
    return h3.reshape(n_cry, wfc.shape[1]), o3.reshape(n_cry, 1)


def kernel(atom_fea, nbr_fea, nbr_idx, crystal_atom_idx, pos, W_emb, b_emb,
           Wr1_0, br1_0, Wr2_0, br2_0, tpw_0,
           Wr1_1, br1_1, Wr2_1, br2_1, tpw_1,
           Wr1_2, br1_2, Wr2_2, br2_2, tpw_2,
           W_fc, b_fc, W_out, b_out):
    n, m, nbr = nbr_fea.shape
    e = n * m
    n_cry, apc = crystal_atom_idx.shape

    wcat = jnp.concatenate([Wr1_0, Wr1_1, Wr1_2], axis=1)
    bcat = jnp.concatenate([br1_0, br1_1, br1_2])[None, :]
    eye = jnp.eye(3, dtype=jnp.float32)
    v = jnp.concatenate([
        _C0 * Wr2_0[:, :1] * eye[0], _C0 * Wr2_1[:, :1] * eye[1],
        _C0 * Wr2_2[:, :1] * eye[2]], axis=0)
    v = jnp.concatenate([v, jnp.zeros((3 * nbr, 1), jnp.float32)], axis=1)
    bv = jnp.stack([_C0 * br2_0[0], _C0 * br2_1[0], _C0 * br2_2[0],
                    jnp.float32(1.0)])[None, :]

    def _blockdiag2(a):
        z = jnp.zeros_like(a)
        return jnp.concatenate([jnp.concatenate([a, z], axis=1),
                                jnp.concatenate([z, a], axis=1)], axis=0)

    wcat2 = _blockdiag2(wcat)
    bcat2 = jnp.concatenate([bcat, bcat], axis=1)
    v2 = _blockdiag2(v)
    bv2 = jnp.concatenate([bv, bv], axis=1)

    radial256 = nbr_fea.reshape(n, m * nbr)
    vals = _edge_scalars(radial256, wcat2, bcat2, v2, bv2,
                         block=5000)

    acc = _sc_scatter(nbr_idx, vals, jnp.zeros((_NPAD * 4,), jnp.float32))
    s_pair = acc[:, :n * 4].reshape(2, n, 4)
    s0 = s_pair[0]
    s1 = s_pair[1]

    rows = 2000
    n_blocks = n // rows
    cpb = rows // apc
    pool = (jnp.repeat(jnp.eye(cpb, dtype=jnp.float32), apc, axis=1)
            * (1.0 / apc))
    tpw = jnp.stack([tpw_0, tpw_1, tpw_2])
    h, out = _dense(atom_fea, s0, s1, W_emb, b_emb[None, :], tpw, pool,
                    W_fc, b_fc[None, :], W_out, b_out[None, :],
                    rows, n_cry, n_blocks)
    return (out, h)


# speedup vs baseline: 1.2531x; 1.1322x over previous
import functools

import jax
import jax.numpy as jnp
from jax import lax
from jax.experimental import pallas as pl
from jax.experimental.pallas import tpu as pltpu
from jax.experimental.pallas import tpu_sc as plsc

_C0 = 0.28209479177387814

_NC = 2
_NS = 16
_NW = _NC * _NS

_GB = 128

_NPAD = 50176


def _softplus(x):
    return jnp.maximum(x, 0.0) + jnp.log1p(jnp.exp(-jnp.abs(x)))


def _edge_body(rlo_ref, rhi_ref, wcat_ref, bcat_ref, v_ref, bv_ref, out_ref):
    nbr = wcat_ref.shape[0]
    m_slots = rlo_ref.shape[1] // nbr
    for q, ref in ((0, rlo_ref), (1, rhi_ref)):
        for m in range(m_slots):
            a = jnp.dot(ref[:, nbr * m:nbr * (m + 1)], wcat_ref[...],
                        preferred_element_type=jnp.float32) + bcat_ref[...]
            a = _softplus(a)
            out_ref[:, 64 * q + 4 * m:64 * q + 4 * m + 4] = jnp.dot(
                a, v_ref[...], preferred_element_type=jnp.float32) + bv_ref[...]


def _edge_scalars(radial256, wcat, bcat, v, bv, block):
    n, w = radial256.shape
    half_blocks = (n // 2) // block
    return pl.pallas_call(
        _edge_body,
        grid=(half_blocks,),
        in_specs=[
            pl.BlockSpec((block, w), lambda i: (i, 0)),
            pl.BlockSpec((block, w), lambda i: (i + half_blocks, 0)),
            pl.BlockSpec(wcat.shape, lambda i: (0, 0)),
            pl.BlockSpec(bcat.shape, lambda i: (0, 0)),
            pl.BlockSpec(v.shape, lambda i: (0, 0)),
            pl.BlockSpec(bv.shape, lambda i: (0, 0)),
        ],
        out_specs=pl.BlockSpec((block, 128), lambda i: (i, 0)),
        out_shape=jax.ShapeDtypeStruct((n // 2, 128), jnp.float32),
    )(radial256, radial256, wcat, bcat, v, bv)


_CHUNK_R = 100
_BIG_W = 30
_BIG_R = 800
_SMALL_R = 500
_SUB = 25


def _sc_scatter(nbr_idx, vals, zeros):
    m = nbr_idx.shape[1]
    half = vals.shape[0]
    acc_w = _NPAD * 4
    words_t = acc_w // _NS

    mesh = plsc.VectorSubcoreMesh(core_axis_name="c", subcore_axis_name="s")

    @functools.partial(
        pl.kernel,
        out_type=jax.ShapeDtypeStruct((_NC, acc_w), jnp.float32),
        mesh=mesh,
        scratch_types=[
            pltpu.VMEM((_CHUNK_R, m), jnp.int32),
            pltpu.VMEM((_CHUNK_R, m), jnp.int32),
            pltpu.VMEM((_CHUNK_R, 128), jnp.int32),
            pltpu.VMEM((_CHUNK_R, 128), jnp.float32),
            pltpu.VMEM_SHARED((acc_w,), jnp.float32),
            pltpu.SemaphoreType.DMA,
        ],
        compiler_params=pltpu.CompilerParams(use_tc_tiling_on_sc=False),
    )
    def scatter(idx_hbm, vals_hbm, zeros_hbm, out_hbm,
                idx_lo, idx_hi, idxw, vals_v, acc, sem):
        c = lax.axis_index("c")
        s = lax.axis_index("s")
        wid = s * _NC + c

        pltpu.sync_copy(zeros_hbm.at[pl.ds(s * words_t, words_t)],
                        acc.at[pl.ds(s * words_t, words_t)])
        plsc.subcore_barrier()

        base = jnp.where(wid < _BIG_W, wid * _BIG_R,
                         _BIG_W * _BIG_R + (wid - _BIG_W) * _SMALL_R)
        n_chunks = jnp.where(wid < _BIG_W, _BIG_R // _CHUNK_R,
                             _SMALL_R // _CHUNK_R)

        lanes = lax.iota(jnp.int32, 16)
        mod4 = lax.rem(lanes, jnp.int32(4))
        div4 = lax.div(lanes, jnp.int32(4))
        dn = lax.GatherDimensionNumbers(
            offset_dims=(), collapsed_slice_dims=(0,), start_index_map=(0,))
        perms = [(div4 + 4 * j)[:, None] for j in range(4)]

        @pl.loop(0, n_chunks)
        def _(k):
            r0 = base + k * _CHUNK_R
            pltpu.sync_copy(idx_hbm.at[pl.ds(r0, _CHUNK_R)], idx_lo)
            pltpu.sync_copy(idx_hbm.at[pl.ds(half + r0, _CHUNK_R)], idx_hi)
            pltpu.sync_copy(vals_hbm.at[pl.ds(r0, _CHUNK_R)], vals_v)

            @pl.loop(0, _CHUNK_R)
            def _(r):
                row_lo = idx_lo[r, :]
                row_hi = idx_hi[r, :]
                for j in range(4):
                    seg = lax.gather(row_lo, perms[j], dn, (1,),
                                     mode=lax.GatherScatterMode.PROMISE_IN_BOUNDS)
                    idxw[r, 16 * j:16 * j + 16] = seg * 4 + mod4
                for j in range(4):
                    seg = lax.gather(row_hi, perms[j], dn, (1,),
                                     mode=lax.GatherScatterMode.PROMISE_IN_BOUNDS)
                    idxw[r, 64 + 16 * j:64 + 16 * j + 16] = seg * 4 + mod4

            for sub in range(_CHUNK_R // _SUB):
                @pl.loop(sub * _SUB, (sub + 1) * _SUB)
                def _(r):
                    pltpu.async_copy(vals_v.at[r], acc.at[idxw.at[r]], sem,
                                     add=True)

                @pl.loop(0, _SUB)
                def _(r):
                    pltpu.make_async_copy(vals_v.at[0], acc.at[idxw.at[0]],
                                          sem).wait()

        plsc.subcore_barrier()
        pltpu.sync_copy(acc.at[pl.ds(s * words_t, words_t)],
                        out_hbm.at[c, pl.ds(s * words_t, words_t)])

    return scatter(nbr_idx, vals, zeros)


def _dense_body(atom_ref, s0_ref, s1_ref, wemb_ref, bemb_ref,
                tpw_ref, pool_ref, wfc_ref, bfc_ref, wout_ref, bout_ref,
                h_ref, o_ref):
    x = jnp.dot(atom_ref[...], wemb_ref[...],
                preferred_element_type=jnp.float32) + bemb_ref[...]
    s = s0_ref[...] + s1_ref[...]
    inv = 1.0 / jnp.maximum(s[:, 3:4], 1.0)
    for j in range(3):
        x = jnp.dot(x * (s[:, j:j + 1] * inv), tpw_ref[j],
                    preferred_element_type=jnp.float32) * 0.125
    crys = jnp.dot(pool_ref[...], x, preferred_element_type=jnp.float32)
    h = _softplus(jnp.dot(crys, wfc_ref[...],
                          preferred_element_type=jnp.float32) + bfc_ref[...])
    h_ref[0] = h
    o_ref[0] = jnp.dot(h, wout_ref[...],
                       preferred_element_type=jnp.float32) + bout_ref[...]


def _dense(atom_fea, s0, s1, wemb, bemb, tpw, pool, wfc, bfc, wout, bout,
           rows, n_cry, n_blocks):
    cpb = n_cry // n_blocks
    h3, o3 = pl.pallas_call(
        _dense_body,
        grid=(n_blocks,),
        in_specs=[
            pl.BlockSpec((rows, atom_fea.shape[1]), lambda i: (i, 0)),
            pl.BlockSpec((rows, 4), lambda i: (i, 0)),
            pl.BlockSpec((rows, 4), lambda i: (i, 0)),
            pl.BlockSpec(wemb.shape, lambda i: (0, 0)),
            pl.BlockSpec(bemb.shape, lambda i: (0, 0)),
            pl.BlockSpec(tpw.shape, lambda i: (0, 0, 0)),
            pl.BlockSpec(pool.shape, lambda i: (0, 0)),
            pl.BlockSpec(wfc.shape, lambda i: (0, 0)),
            pl.BlockSpec(bfc.shape, lambda i: (0, 0)),
            pl.BlockSpec(wout.shape, lambda i: (0, 0)),
            pl.BlockSpec(bout.shape, lambda i: (0, 0)),
        ],
        out_specs=[
            pl.BlockSpec((1, cpb, wfc.shape[1]), lambda i: (i, 0, 0)),
            pl.BlockSpec((1, cpb, 1), lambda i: (i, 0, 0)),
        ],
        out_shape=[
            jax.ShapeDtypeStruct((n_blocks, cpb, wfc.shape[1]), jnp.float32),
            jax.ShapeDtypeStruct((n_blocks, cpb, 1), jnp.float32),
        ],
    )(atom_fea, s0, s1, wemb, bemb, tpw, pool, wfc, bfc, wout, bout)
    return h3.reshape(n_cry, wfc.shape[1]), o3.reshape(n_cry, 1)


def kernel(atom_fea, nbr_fea, nbr_idx, crystal_atom_idx, pos, W_emb, b_emb,
           Wr1_0, br1_0, Wr2_0, br2_0, tpw_0,
           Wr1_1, br1_1, Wr2_1, br2_1, tpw_1,
           Wr1_2, br1_2, Wr2_2, br2_2, tpw_2,
           W_fc, b_fc, W_out, b_out):
    n, m, nbr = nbr_fea.shape
    e = n * m
    n_cry, apc = crystal_atom_idx.shape

    wcat = jnp.concatenate([Wr1_0, Wr1_1, Wr1_2], axis=1)
    bcat = jnp.concatenate([br1_0, br1_1, br1_2])[None, :]
    eye = jnp.eye(3, dtype=jnp.float32)
    v = jnp.concatenate([
        _C0 * Wr2_0[:, :1] * eye[0], _C0 * Wr2_1[:, :1] * eye[1],
        _C0 * Wr2_2[:, :1] * eye[2]], axis=0)
    v = jnp.concatenate([v, jnp.zeros((3 * nbr, 1), jnp.float32)], axis=1)
    bv = jnp.stack([_C0 * br2_0[0], _C0 * br2_1[0], _C0 * br2_2[0],
                    jnp.float32(1.0)])[None, :]

    radial256 = nbr_fea.reshape(n, m * nbr)
    vals = _edge_scalars(radial256, wcat, bcat, v, bv, block=1000)

    acc = _sc_scatter(nbr_idx, vals, jnp.zeros((_NPAD * 4,), jnp.float32))
    s_pair = acc[:, :n * 4].reshape(2, n, 4)
    s0 = s_pair[0]
    s1 = s_pair[1]

    rows = 2000
    n_blocks = n // rows
    cpb = rows // apc
    pool = (jnp.repeat(jnp.eye(cpb, dtype=jnp.float32), apc, axis=1)
            * (1.0 / apc))
    tpw = jnp.stack([tpw_0, tpw_1, tpw_2])
    h, out = _dense(atom_fea, s0, s1, W_emb, b_emb[None, :], tpw, pool,
                    W_fc, b_fc[None, :], W_out, b_out[None, :],
                    rows, n_cry, n_blocks)
    return (out, h)
